# baseline (speedup 1.0000x reference)
import jax
import jax.numpy as jnp
from jax.experimental import pallas as pl
from jax.experimental.pallas import tpu as pltpu

CHUNK = 512
RING = 8


def _router_kernel(x_hbm, w_ref, noise_ref, out_ref, xbuf, sems):
    nsteps = pl.num_programs(0)
    i = pl.program_id(0)
    slot = jax.lax.rem(i, RING)

    def copy_for(step, dst_slot):
        return pltpu.make_async_copy(
            x_hbm.at[pl.ds(step * CHUNK, CHUNK), :],
            xbuf.at[dst_slot],
            sems.at[dst_slot],
        )

    @pl.when(i == 0)
    def _prologue():
        for k in range(RING):
            copy_for(k, k).start()

    copy_for(i, slot).wait()

    logits = jax.lax.dot_general(
        xbuf[slot],
        w_ref[...],
        dimension_numbers=(((1,), (1,)), ((), ())),
        preferred_element_type=jnp.float32,
    )
    out_ref[...] = logits + noise_ref[...]

    @pl.when(i + RING < nsteps)
    def _refill():
        copy_for(i + RING, slot).start()


def kernel(x, W, noise):
    tokens, d_model = x.shape
    n_experts = W.shape[0]
    grid = (tokens // CHUNK,)
    return pl.pallas_call(
        _router_kernel,
        grid=grid,
        in_specs=[
            pl.BlockSpec(memory_space=pltpu.MemorySpace.HBM),
            pl.BlockSpec((n_experts, d_model), lambda i: (0, 0)),
            pl.BlockSpec((CHUNK, n_experts), lambda i: (i, 0)),
        ],
        out_specs=pl.BlockSpec((CHUNK, n_experts), lambda i: (i, 0)),
        out_shape=jax.ShapeDtypeStruct((tokens, n_experts), jnp.float32),
        scratch_shapes=[
            pltpu.VMEM((RING, CHUNK, d_model), jnp.float32),
            pltpu.SemaphoreType.DMA((RING,)),
        ],
        compiler_params=pltpu.CompilerParams(
            dimension_semantics=("arbitrary",),
        ),
    )(x, W, noise)


# speedup vs baseline: 1.1612x; 1.1612x over previous
import jax
import jax.numpy as jnp
from jax.experimental import pallas as pl
from jax.experimental.pallas import tpu as pltpu

TOKEN_BLOCK = 1024
N_SPLITS = 8
CHUNK = TOKEN_BLOCK // N_SPLITS


def _router_kernel(*refs):
    x_refs = refs[:N_SPLITS]
    w_ref, noise_ref, out_ref = refs[N_SPLITS:]
    w = w_ref[...]
    for j, x_ref in enumerate(x_refs):
        logits = jax.lax.dot_general(
            x_ref[...],
            w,
            dimension_numbers=(((1,), (1,)), ((), ())),
            preferred_element_type=jnp.float32,
        )
        rows = pl.ds(j * CHUNK, CHUNK)
        out_ref[rows, :] = logits + noise_ref[rows, :]


def kernel(x, W, noise):
    tokens, d_model = x.shape
    n_experts = W.shape[0]
    grid = (tokens // TOKEN_BLOCK,)

    def x_spec(j):
        return pl.BlockSpec((CHUNK, d_model), lambda i, j=j: (N_SPLITS * i + j, 0))

    return pl.pallas_call(
        _router_kernel,
        grid=grid,
        in_specs=[x_spec(j) for j in range(N_SPLITS)] + [
            pl.BlockSpec((n_experts, d_model), lambda i: (0, 0)),
            pl.BlockSpec((TOKEN_BLOCK, n_experts), lambda i: (i, 0)),
        ],
        out_specs=pl.BlockSpec((TOKEN_BLOCK, n_experts), lambda i: (i, 0)),
        out_shape=jax.ShapeDtypeStruct((tokens, n_experts), jnp.float32),
        compiler_params=pltpu.CompilerParams(
            dimension_semantics=("arbitrary",),
        ),
    )(*([x] * N_SPLITS), W, noise)


# speedup vs baseline: 1.4793x; 1.2739x over previous
import jax
import jax.numpy as jnp
from jax.experimental import pallas as pl
from jax.experimental.pallas import tpu as pltpu

TOKEN_BLOCK = 1024


def _router_kernel(x_ref, w_ref, noise_ref, out_ref):
    logits_t = jax.lax.dot_general(
        w_ref[...],
        x_ref[...],
        dimension_numbers=(((1,), (1,)), ((), ())),
        preferred_element_type=jnp.float32,
    )
    out_ref[...] = logits_t + noise_ref[...]


def kernel(x, W, noise):
    tokens, d_model = x.shape
    n_experts = W.shape[0]
    noise_t = noise.T
    grid = (tokens // TOKEN_BLOCK,)
    out_t = pl.pallas_call(
        _router_kernel,
        grid=grid,
        in_specs=[
            pl.BlockSpec((TOKEN_BLOCK, d_model), lambda i: (i, 0)),
            pl.BlockSpec((n_experts, d_model), lambda i: (0, 0)),
            pl.BlockSpec((n_experts, TOKEN_BLOCK), lambda i: (0, i)),
        ],
        out_specs=pl.BlockSpec((n_experts, TOKEN_BLOCK), lambda i: (0, i)),
        out_shape=jax.ShapeDtypeStruct((n_experts, tokens), jnp.float32),
        compiler_params=pltpu.CompilerParams(
            dimension_semantics=("arbitrary",),
        ),
    )(x, W, noise_t)
    return out_t.T
